# baseline (speedup 1.0000x reference)
import functools

import jax
import jax.numpy as jnp
from jax import lax
from jax.experimental import pallas as pl
from jax.experimental.pallas import tpu as pltpu
from jax.experimental.pallas import tpu_sc as plsc

_G = 512
_R = 192
_NW = 32


def _score_body(batch_ref, x_ref, Wp_ref, bp_ref, Ws_ref, bs_ref,
                e_ref, den_ref, offs_ref, cnt_ref, dacc):
    i = pl.program_id(0)
    k = pl.num_programs(0)
    x = x_ref[...]
    ht = jnp.tanh(
        lax.dot_general(Wp_ref[...].astype(jnp.bfloat16),
                        x.astype(jnp.bfloat16),
                        (((0,), (1,)), ((), ())),
                        preferred_element_type=jnp.float32)
        + bp_ref[...])
    s = jnp.dot(Ws_ref[...], ht, preferred_element_type=jnp.float32) \
        + bs_ref[0, 0]
    e = jnp.exp(s)
    e_ref[...] = e.reshape(1, 1, e.shape[1])

    bids = batch_ref[0]
    gi = lax.broadcasted_iota(jnp.int32, (_G, bids.shape[1]), 0)
    onehot = (gi == bids).astype(jnp.float32)
    y2 = jnp.concatenate([e, jnp.ones_like(e)], axis=0)
    dc = lax.dot_general(y2, onehot, (((1,), (1,)), ((), ())),
                         preferred_element_type=jnp.float32)

    @pl.when(i == 0)
    def _init():
        dacc[...] = dc

    @pl.when(i > 0)
    def _acc():
        dacc[...] += dc

    @pl.when(i == k - 1)
    def _emit():
        den_ref[...] = dacc[0:1, :]
        cnt = dacc[1:2, :]
        ri = lax.broadcasted_iota(jnp.int32, (_G, _G), 0)
        ci = lax.broadcasted_iota(jnp.int32, (_G, _G), 1)
        tri = (ci > ri).astype(jnp.float32)
        offs = lax.dot_general(cnt, tri, (((1,), (0,)), ((), ())),
                               preferred_element_type=jnp.float32)
        offs_ref[...] = offs.astype(jnp.int32)
        cnt_ref[...] = cnt.astype(jnp.int32)


def _scores(x, batch, Wp, bp, Ws, bs):
    n, d = x.shape
    blk = 10000
    k = n // blk
    assert k * blk == n
    batch3 = batch.reshape(k, 1, blk)
    return pl.pallas_call(
        _score_body,
        grid=(k,),
        in_specs=[
            pl.BlockSpec((1, 1, blk), lambda i: (i, 0, 0)),
            pl.BlockSpec((blk, d), lambda i: (i, 0)),
            pl.BlockSpec((d, d), lambda i: (0, 0)),
            pl.BlockSpec((d, 1), lambda i: (0, 0)),
            pl.BlockSpec((1, d), lambda i: (0, 0)),
            pl.BlockSpec((1, 1), lambda i: (0, 0)),
        ],
        out_specs=[
            pl.BlockSpec((1, 1, blk), lambda i: (i, 0, 0)),
            pl.BlockSpec((1, _G), lambda i: (0, 0)),
            pl.BlockSpec((1, _G), lambda i: (0, 0)),
            pl.BlockSpec((1, _G), lambda i: (0, 0)),
        ],
        out_shape=[
            jax.ShapeDtypeStruct((k, 1, blk), jnp.float32),
            jax.ShapeDtypeStruct((1, _G), jnp.float32),
            jax.ShapeDtypeStruct((1, _G), jnp.int32),
            jax.ShapeDtypeStruct((1, _G), jnp.int32),
        ],
        scratch_shapes=[pltpu.VMEM((2, _G), jnp.float32)],
    )(batch3, x, Wp, bp.reshape(d, 1), Ws.reshape(1, d), bs.reshape(1, 1))


def _sc_pool_body(x_hbm, batch_hbm, e_hbm, den_hbm, offs_hbm, cnt_hbm,
                  out_hbm, xv, ev, bv, xv2, ev2, bv2, offs_v, cnt_v, inv_v,
                  acc, sem1, sem2, sem3, sem4, sem5, sem6):
    n, d = x_hbm.shape
    nc16 = d // 16
    wid = lax.axis_index("s") * 2 + lax.axis_index("c")
    base_seg = wid * 16

    pltpu.sync_copy(offs_hbm.at[pl.ds(base_seg, 16)], offs_v)
    pltpu.sync_copy(cnt_hbm.at[pl.ds(base_seg, 16)], cnt_v)
    pltpu.sync_copy(den_hbm.at[pl.ds(base_seg, 16)], inv_v.at[pl.ds(0, 16)])
    inv_v[pl.ds(0, 16)] = 1.0 / jnp.maximum(inv_v[pl.ds(0, 16)], 1e-37)

    for g in range(16):
        for c in range(nc16):
            acc[g, pl.ds(c * 16, 16)] = jnp.zeros((16,), jnp.float32)

    offs_vec = offs_v[...]
    cnt_vec = cnt_v[...]
    start_w = offs_vec[0]
    end_w = offs_vec[15] + cnt_vec[15]

    def _base8(ci):
        row = start_w + ci * _R
        return row, jnp.minimum((row // 8) * 8, n - (_R + 8))

    def _issue(ci, xb, eb, bb, sx, se, sb):
        _, base8 = _base8(ci)
        pltpu.async_copy(x_hbm.at[pl.ds(base8, _R + 8)], xb, sx)
        pltpu.async_copy(e_hbm.at[pl.ds(base8, _R + 8)],
                         eb.at[pl.ds(0, _R + 8)], se)
        pltpu.async_copy(batch_hbm.at[pl.ds(base8, _R + 8)],
                         bb.at[pl.ds(0, _R + 8)], sb)

    def _wait(ci, xb, eb, bb, sx, se, sb):
        _, base8 = _base8(ci)
        pltpu.make_async_copy(x_hbm.at[pl.ds(base8, _R + 8)], xb, sx).wait()
        pltpu.make_async_copy(e_hbm.at[pl.ds(base8, _R + 8)],
                              eb.at[pl.ds(0, _R + 8)], se).wait()
        pltpu.make_async_copy(batch_hbm.at[pl.ds(base8, _R + 8)],
                              bb.at[pl.ds(0, _R + 8)], sb).wait()

    def _compute(ci, xb, eb, bb):
        row, base8 = _base8(ci)
        de = row - base8
        cnt_chunk = jnp.minimum(_R, end_w - row)

        @plsc.parallel_loop(0, cnt_chunk, unroll=8)
        def _row(r):
            g = bb[pl.ds(de + r, 16)][0] - base_seg
            w = eb[pl.ds(de + r, 16)][0]
            for c in range(nc16):
                plsc.addupdate(acc.at[g, pl.ds(c * 16, 16)],
                               w * xb[de + r, pl.ds(c * 16, 16)])

    def _pair(i, carry):
        _wait(2 * i, xv, ev, bv, sem1, sem2, sem3)
        _issue(2 * i + 1, xv2, ev2, bv2, sem4, sem5, sem6)
        _compute(2 * i, xv, ev, bv)
        _wait(2 * i + 1, xv2, ev2, bv2, sem4, sem5, sem6)
        _issue(2 * i + 2, xv, ev, bv, sem1, sem2, sem3)
        _compute(2 * i + 1, xv2, ev2, bv2)
        return carry

    n_chunks = (end_w - start_w + _R - 1) // _R
    n_pairs = (n_chunks + 1) // 2
    _issue(0, xv, ev, bv, sem1, sem2, sem3)
    lax.fori_loop(0, n_pairs, _pair, 0)
    _wait(2 * n_pairs, xv, ev, bv, sem1, sem2, sem3)

    for g in range(16):
        iv = inv_v[pl.ds(g, 16)][0]
        for c in range(nc16):
            acc[g, pl.ds(c * 16, 16)] = acc[g, pl.ds(c * 16, 16)] * iv
    pltpu.sync_copy(acc, out_hbm.at[pl.ds(base_seg, 16)])


def _sc_pool(x, batch, e, den, offs, cnt):
    n, d = x.shape
    mesh = plsc.VectorSubcoreMesh(core_axis_name="c", subcore_axis_name="s")
    f = functools.partial(
        pl.kernel,
        out_type=jax.ShapeDtypeStruct((_G, d), jnp.float32),
        mesh=mesh,
        scratch_types=[
            pltpu.VMEM((_R + 8, d), jnp.float32),
            pltpu.VMEM((_R + 24,), jnp.float32),
            pltpu.VMEM((_R + 24,), jnp.int32),
            pltpu.VMEM((_R + 8, d), jnp.float32),
            pltpu.VMEM((_R + 24,), jnp.float32),
            pltpu.VMEM((_R + 24,), jnp.int32),
            pltpu.VMEM((16,), jnp.int32),
            pltpu.VMEM((16,), jnp.int32),
            pltpu.VMEM((32,), jnp.float32),
            pltpu.VMEM((16, d), jnp.float32),
            pltpu.SemaphoreType.DMA,
            pltpu.SemaphoreType.DMA,
            pltpu.SemaphoreType.DMA,
            pltpu.SemaphoreType.DMA,
            pltpu.SemaphoreType.DMA,
            pltpu.SemaphoreType.DMA,
        ],
    )(_sc_pool_body)
    return f(x, batch, e, den, offs, cnt)


def kernel(x, batch, Wp, bp, Ws, bs):
    n, d = x.shape
    e, den, offs, cnt = _scores(x, batch, Wp, bp, Ws, bs)
    return _sc_pool(x, batch, e.reshape(n), den.reshape(_G),
                    offs.reshape(_G), cnt.reshape(_G))


# speedup vs baseline: 1.0212x; 1.0212x over previous
import functools

import jax
import jax.numpy as jnp
from jax import lax
from jax.experimental import pallas as pl
from jax.experimental.pallas import tpu as pltpu
from jax.experimental.pallas import tpu_sc as plsc

_G = 512
_R = 128
_NW = 32


def _score_body(batch_ref, x_ref, Wp_ref, bp_ref, Ws_ref, bs_ref,
                e_ref, den_ref, offs_ref, cnt_ref, dacc):
    i = pl.program_id(0)
    k = pl.num_programs(0)
    x = x_ref[...]
    ht = jnp.tanh(
        lax.dot_general(Wp_ref[...].astype(jnp.bfloat16),
                        x.astype(jnp.bfloat16),
                        (((0,), (1,)), ((), ())),
                        preferred_element_type=jnp.float32)
        + bp_ref[...])
    s = jnp.dot(Ws_ref[...], ht, preferred_element_type=jnp.float32) \
        + bs_ref[0, 0]
    e = jnp.exp(s)
    e_ref[...] = e.reshape(1, 1, e.shape[1])

    bids = batch_ref[0]
    gi = lax.broadcasted_iota(jnp.int32, (_G, bids.shape[1]), 0)
    onehot = (gi == bids).astype(jnp.float32)
    y2 = jnp.concatenate([e, jnp.ones_like(e)], axis=0)
    dc = lax.dot_general(y2, onehot, (((1,), (1,)), ((), ())),
                         preferred_element_type=jnp.float32)

    @pl.when(i == 0)
    def _init():
        dacc[...] = dc

    @pl.when(i > 0)
    def _acc():
        dacc[...] += dc

    @pl.when(i == k - 1)
    def _emit():
        den_ref[...] = dacc[0:1, :]
        cnt = dacc[1:2, :]
        ri = lax.broadcasted_iota(jnp.int32, (_G, _G), 0)
        ci = lax.broadcasted_iota(jnp.int32, (_G, _G), 1)
        tri = (ci > ri).astype(jnp.float32)
        offs = lax.dot_general(cnt, tri, (((1,), (0,)), ((), ())),
                               preferred_element_type=jnp.float32)
        offs_ref[...] = offs.astype(jnp.int32)
        cnt_ref[...] = cnt.astype(jnp.int32)


def _scores(x, batch, Wp, bp, Ws, bs):
    n, d = x.shape
    blk = 10000
    k = n // blk
    assert k * blk == n
    batch3 = batch.reshape(k, 1, blk)
    return pl.pallas_call(
        _score_body,
        grid=(k,),
        in_specs=[
            pl.BlockSpec((1, 1, blk), lambda i: (i, 0, 0)),
            pl.BlockSpec((blk, d), lambda i: (i, 0)),
            pl.BlockSpec((d, d), lambda i: (0, 0)),
            pl.BlockSpec((d, 1), lambda i: (0, 0)),
            pl.BlockSpec((1, d), lambda i: (0, 0)),
            pl.BlockSpec((1, 1), lambda i: (0, 0)),
        ],
        out_specs=[
            pl.BlockSpec((1, 1, blk), lambda i: (i, 0, 0)),
            pl.BlockSpec((1, _G), lambda i: (0, 0)),
            pl.BlockSpec((1, _G), lambda i: (0, 0)),
            pl.BlockSpec((1, _G), lambda i: (0, 0)),
        ],
        out_shape=[
            jax.ShapeDtypeStruct((k, 1, blk), jnp.float32),
            jax.ShapeDtypeStruct((1, _G), jnp.float32),
            jax.ShapeDtypeStruct((1, _G), jnp.int32),
            jax.ShapeDtypeStruct((1, _G), jnp.int32),
        ],
        scratch_shapes=[pltpu.VMEM((2, _G), jnp.float32)],
    )(batch3, x, Wp, bp.reshape(d, 1), Ws.reshape(1, d), bs.reshape(1, 1))


def _sc_pool_body(x_hbm, batch_hbm, e_hbm, den_hbm, offs_hbm, cnt_hbm,
                  out_hbm, xv, ev, bv, xv2, ev2, bv2, offs_v, cnt_v, inv_v,
                  acc, sem1, sem2, sem3, sem4, sem5, sem6):
    n, d = x_hbm.shape
    nc16 = d // 16
    wid = lax.axis_index("s") * 2 + lax.axis_index("c")
    base_seg = wid * 16

    pltpu.sync_copy(offs_hbm.at[pl.ds(base_seg, 16)], offs_v)
    pltpu.sync_copy(cnt_hbm.at[pl.ds(base_seg, 16)], cnt_v)
    pltpu.sync_copy(den_hbm.at[pl.ds(base_seg, 16)], inv_v.at[pl.ds(0, 16)])
    inv_v[pl.ds(0, 16)] = 1.0 / jnp.maximum(inv_v[pl.ds(0, 16)], 1e-37)

    for g in range(16):
        for c in range(nc16):
            acc[g, pl.ds(c * 16, 16)] = jnp.zeros((16,), jnp.float32)

    offs_vec = offs_v[...]
    cnt_vec = cnt_v[...]
    start_w = offs_vec[0]
    end_w = offs_vec[15] + cnt_vec[15]

    def _base8(ci):
        row = start_w + ci * _R
        return row, jnp.minimum((row // 8) * 8, n - (_R + 8))

    def _issue(ci, xb, eb, bb, sx, se, sb):
        _, base8 = _base8(ci)
        pltpu.async_copy(x_hbm.at[pl.ds(base8, _R + 8)], xb, sx)
        pltpu.async_copy(e_hbm.at[pl.ds(base8, _R + 8)],
                         eb.at[pl.ds(0, _R + 8)], se)
        pltpu.async_copy(batch_hbm.at[pl.ds(base8, _R + 8)],
                         bb.at[pl.ds(0, _R + 8)], sb)

    def _wait(ci, xb, eb, bb, sx, se, sb):
        _, base8 = _base8(ci)
        pltpu.make_async_copy(x_hbm.at[pl.ds(base8, _R + 8)], xb, sx).wait()
        pltpu.make_async_copy(e_hbm.at[pl.ds(base8, _R + 8)],
                              eb.at[pl.ds(0, _R + 8)], se).wait()
        pltpu.make_async_copy(batch_hbm.at[pl.ds(base8, _R + 8)],
                              bb.at[pl.ds(0, _R + 8)], sb).wait()

    def _compute(ci, xb, eb, bb):
        row, base8 = _base8(ci)
        de = row - base8
        cnt_chunk = jnp.minimum(_R, end_w - row)

        @plsc.parallel_loop(0, cnt_chunk, unroll=8)
        def _row(r):
            g = bb[pl.ds(de + r, 16)][0] - base_seg
            w = eb[pl.ds(de + r, 16)][0]
            for c in range(nc16):
                plsc.addupdate(acc.at[g, pl.ds(c * 16, 16)],
                               w * xb[de + r, pl.ds(c * 16, 16)])

    def _pair(i, carry):
        _wait(2 * i, xv, ev, bv, sem1, sem2, sem3)
        _issue(2 * i + 1, xv2, ev2, bv2, sem4, sem5, sem6)
        _compute(2 * i, xv, ev, bv)
        _wait(2 * i + 1, xv2, ev2, bv2, sem4, sem5, sem6)
        _issue(2 * i + 2, xv, ev, bv, sem1, sem2, sem3)
        _compute(2 * i + 1, xv2, ev2, bv2)
        return carry

    n_chunks = (end_w - start_w + _R - 1) // _R
    n_pairs = (n_chunks + 1) // 2
    _issue(0, xv, ev, bv, sem1, sem2, sem3)
    lax.fori_loop(0, n_pairs, _pair, 0)
    _wait(2 * n_pairs, xv, ev, bv, sem1, sem2, sem3)

    for g in range(16):
        iv = inv_v[pl.ds(g, 16)][0]
        for c in range(nc16):
            acc[g, pl.ds(c * 16, 16)] = acc[g, pl.ds(c * 16, 16)] * iv
    pltpu.sync_copy(acc, out_hbm.at[pl.ds(base_seg, 16)])


def _sc_pool(x, batch, e, den, offs, cnt):
    n, d = x.shape
    mesh = plsc.VectorSubcoreMesh(core_axis_name="c", subcore_axis_name="s")
    f = functools.partial(
        pl.kernel,
        out_type=jax.ShapeDtypeStruct((_G, d), jnp.float32),
        mesh=mesh,
        scratch_types=[
            pltpu.VMEM((_R + 8, d), jnp.float32),
            pltpu.VMEM((_R + 24,), jnp.float32),
            pltpu.VMEM((_R + 24,), jnp.int32),
            pltpu.VMEM((_R + 8, d), jnp.float32),
            pltpu.VMEM((_R + 24,), jnp.float32),
            pltpu.VMEM((_R + 24,), jnp.int32),
            pltpu.VMEM((16,), jnp.int32),
            pltpu.VMEM((16,), jnp.int32),
            pltpu.VMEM((32,), jnp.float32),
            pltpu.VMEM((16, d), jnp.float32),
            pltpu.SemaphoreType.DMA,
            pltpu.SemaphoreType.DMA,
            pltpu.SemaphoreType.DMA,
            pltpu.SemaphoreType.DMA,
            pltpu.SemaphoreType.DMA,
            pltpu.SemaphoreType.DMA,
        ],
    )(_sc_pool_body)
    return f(x, batch, e, den, offs, cnt)


def kernel(x, batch, Wp, bp, Ws, bs):
    n, d = x.shape
    e, den, offs, cnt = _scores(x, batch, Wp, bp, Ws, bs)
    return _sc_pool(x, batch, e.reshape(n), den.reshape(_G),
                    offs.reshape(_G), cnt.reshape(_G))
